# baseline (speedup 1.0000x reference)
import functools

import jax
import jax.numpy as jnp
from jax import lax
from jax.experimental import pallas as pl
from jax.experimental.pallas import tpu as pltpu
from jax.experimental.pallas import tpu_sc as plsc

VOCAB = 1000000
EMBED = 64
BATCH = 4096

_NC = 2
_NS = 16
_NW = _NC * _NS
_BPW = BATCH // _NW

_mesh = plsc.VectorSubcoreMesh(core_axis_name="c", subcore_axis_name="s")


@functools.partial(
    pl.kernel,
    mesh=_mesh,
    compiler_params=pltpu.CompilerParams(use_tc_tiling_on_sc=False),
    out_type=[
        jax.ShapeDtypeStruct((BATCH, EMBED), jnp.float32),
        jax.ShapeDtypeStruct((BATCH, EMBED), jnp.float32),
    ],
    scratch_types=[
        pltpu.VMEM((_BPW,), jnp.int32),
        pltpu.VMEM((_BPW,), jnp.int32),
        pltpu.VMEM((_BPW, EMBED), jnp.float32),
        pltpu.VMEM((_BPW, EMBED), jnp.float32),
        pltpu.SemaphoreType.DMA,
        pltpu.SemaphoreType.DMA,
    ],
)
def _sc_gather(tgt_hbm, ctx_hbm, in_tab, out_tab, in_rows_hbm, out_rows_hbm,
               idx_t, idx_c, rows_t, rows_c, sem_t, sem_c):
    wid = lax.axis_index("s") * _NC + lax.axis_index("c")
    base = wid * _BPW
    pltpu.sync_copy(tgt_hbm.at[pl.ds(base, _BPW)], idx_t)
    pltpu.sync_copy(ctx_hbm.at[pl.ds(base, _BPW)], idx_c)
    cp_t = pltpu.async_copy(in_tab.at[idx_t], rows_t, sem_t)
    cp_c = pltpu.async_copy(out_tab.at[idx_c], rows_c, sem_c)
    cp_t.wait()
    cp_c.wait()
    pltpu.sync_copy(rows_t, in_rows_hbm.at[pl.ds(base, _BPW)])
    pltpu.sync_copy(rows_c, out_rows_hbm.at[pl.ds(base, _BPW)])


_BM = 256


def _matmul_body(a_ref, b_ref, o_ref):
    o_ref[...] = lax.dot_general(
        a_ref[...], b_ref[...],
        (((1,), (1,)), ((), ())),
        preferred_element_type=jnp.float32,
    )


_matmul = pl.pallas_call(
    _matmul_body,
    grid=(BATCH // _BM,),
    in_specs=[
        pl.BlockSpec((_BM, EMBED), lambda i: (i, 0)),
        pl.BlockSpec((BATCH, EMBED), lambda i: (0, 0)),
    ],
    out_specs=pl.BlockSpec((_BM, BATCH), lambda i: (i, 0)),
    out_shape=jax.ShapeDtypeStruct((BATCH, BATCH), jnp.float32),
)


def kernel(target, context, in_embed, out_embed):
    target = target.astype(jnp.int32)
    context = context.astype(jnp.int32)
    in_emb, out_emb = _sc_gather(target, context, in_embed, out_embed)
    return _matmul(in_emb, out_emb)


# speedup vs baseline: 1.5646x; 1.5646x over previous
import functools

import jax
import jax.numpy as jnp
from jax import lax
from jax.experimental import pallas as pl
from jax.experimental.pallas import tpu as pltpu
from jax.experimental.pallas import tpu_sc as plsc

VOCAB = 1000000
EMBED = 64
BATCH = 4096

_NC = 2
_NS = 16
_NW = _NC * _NS
_BPW = BATCH // _NW

_mesh = plsc.VectorSubcoreMesh(core_axis_name="c", subcore_axis_name="s")


@functools.partial(
    pl.kernel,
    mesh=_mesh,
    out_type=[
        jax.ShapeDtypeStruct((BATCH, EMBED), jnp.float32),
        jax.ShapeDtypeStruct((BATCH, EMBED), jnp.float32),
    ],
    scratch_types=[
        pltpu.VMEM((_BPW,), jnp.int32),
        pltpu.VMEM((_BPW,), jnp.int32),
        pltpu.VMEM((_BPW, EMBED), jnp.float32),
        pltpu.VMEM((_BPW, EMBED), jnp.float32),
        pltpu.SemaphoreType.DMA,
        pltpu.SemaphoreType.DMA,
    ],
)
def _sc_gather(tgt_hbm, ctx_hbm, in_tab, out_tab, in_rows_hbm, out_rows_hbm,
               idx_t, idx_c, rows_t, rows_c, sem_t, sem_c):
    wid = lax.axis_index("s") * _NC + lax.axis_index("c")
    base = wid * _BPW
    pltpu.sync_copy(tgt_hbm.at[pl.ds(base, _BPW)], idx_t)
    pltpu.sync_copy(ctx_hbm.at[pl.ds(base, _BPW)], idx_c)

    def fire(g, carry):
        b = g * 16
        vt = idx_t[pl.ds(b, 16)]
        vc = idx_c[pl.ds(b, 16)]
        for j in range(16):
            pltpu.make_async_copy(in_tab.at[vt[j]], rows_t.at[b + j], sem_t).start()
            pltpu.make_async_copy(out_tab.at[vc[j]], rows_c.at[b + j], sem_c).start()
        return carry

    lax.fori_loop(0, _BPW // 16, fire, 0)
    pltpu.make_async_copy(in_tab.at[pl.ds(0, _BPW)], rows_t, sem_t).wait()
    pltpu.make_async_copy(out_tab.at[pl.ds(0, _BPW)], rows_c, sem_c).wait()

    pltpu.sync_copy(rows_t, in_rows_hbm.at[pl.ds(base, _BPW)])
    pltpu.sync_copy(rows_c, out_rows_hbm.at[pl.ds(base, _BPW)])


_BM = 256


def _matmul_body(a_ref, b_ref, o_ref):
    o_ref[...] = lax.dot_general(
        a_ref[...], b_ref[...],
        (((1,), (1,)), ((), ())),
        preferred_element_type=jnp.float32,
    )


_matmul = pl.pallas_call(
    _matmul_body,
    grid=(BATCH // _BM,),
    in_specs=[
        pl.BlockSpec((_BM, EMBED), lambda i: (i, 0)),
        pl.BlockSpec((BATCH, EMBED), lambda i: (0, 0)),
    ],
    out_specs=pl.BlockSpec((_BM, BATCH), lambda i: (i, 0)),
    out_shape=jax.ShapeDtypeStruct((BATCH, BATCH), jnp.float32),
)


def kernel(target, context, in_embed, out_embed):
    target = target.astype(jnp.int32)
    context = context.astype(jnp.int32)
    in_emb, out_emb = _sc_gather(target, context, in_embed, out_embed)
    return _matmul(in_emb, out_emb)
